# Initial kernel scaffold; baseline (speedup 1.0000x reference)
import jax
import jax.numpy as jnp
from jax.experimental import pallas as pl


def kernel(x_user, x_item, edge_index_ui, edge_index_iu, sage_ui_Wl, sage_ui_bl, sage_ui_Wr, sage_iu_Wl, sage_iu_bl, sage_iu_Wr, gat_ui_Wl, gat_ui_bl, gat_ui_Wr, gat_ui_br, gat_ui_att, gat_ui_b, gat_iu_Wl, gat_iu_bl, gat_iu_Wr, gat_iu_br, gat_iu_att, gat_iu_b):
    raise NotImplementedError("write your pallas kernel here")


# speedup vs baseline: 2.9826x; 2.9826x over previous
import functools

import jax
import jax.numpy as jnp
from jax import lax
from jax.experimental import pallas as pl
from jax.experimental.pallas import tpu as pltpu
from jax.experimental.pallas import tpu_sc as plsc

N = 10000
NP = 10240
D = 128
E = 320000
EP = 327680
NC = 2
NS = 16
NW = NC * NS
EW = EP // NW
CH = 128
NCH = EW // CH
NSL = NP // NS
PADN = NP - 1

f32 = jnp.float32
i32 = jnp.int32

_mesh = plsc.VectorSubcoreMesh(
    core_axis_name="c", subcore_axis_name="s", num_cores=NC, num_subcores=NS)


def _zero_vmem(ref, n):
  def b(k, _):
    ref[pl.ds(k * 16, 16)] = jnp.zeros((16,), f32)
    return _
  lax.fori_loop(0, n // 16, b, None)


def _make_seg_sum(weighted):
  def body(table, src, dst, w, zeros_nd,
           acc_out, hist_out,
           sidx, didx, wv, rows, hist, cbuf, ovec, acc_sh, hist_sh, sem):
    c = lax.axis_index("c")
    s = lax.axis_index("s")
    wid = c * NS + s
    pltpu.sync_copy(zeros_nd.at[pl.ds(s * NSL, NSL)],
                    acc_sh.at[pl.ds(s * NSL, NSL)])
    _zero_vmem(hist, NP)
    plsc.subcore_barrier()

    def chunk(ci, _):
      base = wid * EW + ci * CH
      pltpu.sync_copy(src.at[pl.ds(base, CH)], sidx)
      pltpu.sync_copy(dst.at[pl.ds(base, CH)], didx)
      pltpu.async_copy(table.at[sidx], rows, sem).wait()
      if weighted:
        pltpu.sync_copy(w.at[pl.ds(base, CH)], wv)

        def scale(e, __):
          ws = plsc.load_gather(wv, [jnp.full((16,), e, i32)])
          for k in range(D // 16):
            rows[e, pl.ds(k * 16, 16)] = rows[e, pl.ds(k * 16, 16)] * ws
          return __
        lax.fori_loop(0, CH, scale, None)
      pltpu.sync_copy(rows, acc_sh.at[didx], add=True)
      for j in range(CH // 16):
        dv = didx[pl.ds(j * 16, 16)]
        if weighted:
          vals = wv[pl.ds(j * 16, 16)]
        else:
          vals = jnp.ones((16,), f32)
        plsc.addupdate_scatter(hist, [dv], vals)
      return _
    lax.fori_loop(0, NCH, chunk, None)

    pltpu.sync_copy(hist, hist_sh.at[s])
    plsc.subcore_barrier()
    pltpu.sync_copy(acc_sh.at[pl.ds(s * NSL, NSL)],
                    acc_out.at[c, pl.ds(s * NSL, NSL)])
    pltpu.sync_copy(hist_sh.at[:, pl.ds(s * NSL, NSL)], cbuf)

    def comb(k, _):
      v = cbuf[0, pl.ds(k * 16, 16)]
      for r in range(1, NS):
        v = v + cbuf[r, pl.ds(k * 16, 16)]
      ovec[pl.ds(k * 16, 16)] = v
      return _
    lax.fori_loop(0, NSL // 16, comb, None)
    pltpu.sync_copy(ovec, hist_out.at[c, pl.ds(s * NSL, NSL)])

  return pl.kernel(
      body,
      out_type=(jax.ShapeDtypeStruct((NC, NP, D), f32),
                jax.ShapeDtypeStruct((NC, NP), f32)),
      mesh=_mesh,
      scratch_types=[
          pltpu.VMEM((CH,), i32),
          pltpu.VMEM((CH,), i32),
          pltpu.VMEM((CH,), f32),
          pltpu.VMEM((CH, D), f32),
          pltpu.VMEM((NP,), f32),
          pltpu.VMEM((NS, NSL), f32),
          pltpu.VMEM((NSL,), f32),
          pltpu.VMEM_SHARED((NP, D), f32),
          pltpu.VMEM_SHARED((NS, NP), f32),
          pltpu.SemaphoreType.DMA,
      ],
      compiler_params=pltpu.CompilerParams(needs_layout_passes=False),
      name="sc_seg_sum_w" if weighted else "sc_seg_sum",
  )


_seg_sum = _make_seg_sum(False)
_seg_sum_w = _make_seg_sum(True)


def _logits_body(hl, hr, src, dst, att,
                 lo,
                 sidx, didx, abuf, bbuf, attv, lbuf, sem):
  c = lax.axis_index("c")
  s = lax.axis_index("s")
  wid = c * NS + s
  pltpu.sync_copy(att, attv)
  attk = [attv[pl.ds(k * 16, 16)] for k in range(D // 16)]
  lane0 = lax.iota(i32, 16) == 0

  def chunk(ci, _):
    base = wid * EW + ci * CH
    pltpu.sync_copy(src.at[pl.ds(base, CH)], sidx)
    pltpu.sync_copy(dst.at[pl.ds(base, CH)], didx)
    pltpu.async_copy(hl.at[sidx], abuf, sem).wait()
    pltpu.async_copy(hr.at[didx], bbuf, sem).wait()

    def edge(e, __):
      acc = jnp.zeros((16,), f32)
      for k in range(D // 16):
        t = abuf[e, pl.ds(k * 16, 16)] + bbuf[e, pl.ds(k * 16, 16)]
        lr = jnp.maximum(t, 0.2 * t)
        acc = acc + lr * attk[k]
      lg = jnp.sum(acc)
      plsc.store_scatter(lbuf, [jnp.full((16,), e, i32)],
                         jnp.full((16,), lg, f32), mask=lane0)
      return __
    lax.fori_loop(0, CH, edge, None)
    pltpu.sync_copy(lbuf, lo.at[pl.ds(base, CH)])
    return _
  lax.fori_loop(0, NCH, chunk, None)


_logits = pl.kernel(
    _logits_body,
    out_type=jax.ShapeDtypeStruct((EP,), f32),
    mesh=_mesh,
    scratch_types=[
        pltpu.VMEM((CH,), i32),
        pltpu.VMEM((CH,), i32),
        pltpu.VMEM((CH, D), f32),
        pltpu.VMEM((CH, D), f32),
        pltpu.VMEM((D,), f32),
        pltpu.VMEM((CH,), f32),
        pltpu.SemaphoreType.DMA,
    ],
    compiler_params=pltpu.CompilerParams(needs_layout_passes=False),
    name="sc_gat_logits",
)


BM = 1024


def _mm2(x2, w2):
  def body(x_ref, w_ref, o_ref):
    o_ref[0] = jnp.dot(x_ref[0], w_ref[0], preferred_element_type=f32)
  return pl.pallas_call(
      body,
      grid=(2, NP // BM),
      in_specs=[pl.BlockSpec((1, BM, D), lambda a, b: (a, b, 0)),
                pl.BlockSpec((1, D, D), lambda a, b: (a, 0, 0))],
      out_specs=pl.BlockSpec((1, BM, D), lambda a, b: (a, b, 0)),
      out_shape=jax.ShapeDtypeStruct((2, NP, D), f32),
  )(x2, w2)


def _layer_mid(accu, cntu, xu, acci, cnti, xi,
               sWr_iu, sbl_iu, sWr_ui, sbl_ui,
               gWl_iu, gbl_iu, gWr_iu, gbr_iu,
               gWl_ui, gbl_ui, gWr_ui, gbr_ui):
  def body(accu_ref, cntu_ref, xu_ref, acci_ref, cnti_ref, xi_ref,
           sWr_iu_r, sbl_iu_r, sWr_ui_r, sbl_ui_r,
           gWl_iu_r, gbl_iu_r, gWr_iu_r, gbr_iu_r,
           gWl_ui_r, gbl_ui_r, gWr_ui_r, gbr_ui_r,
           hl_iu_r, hr_iu_r, hl_ui_r, hr_ui_r):
    cu = jnp.maximum(cntu_ref[0] + cntu_ref[1], 1.0)
    aggu = (accu_ref[0] + accu_ref[1]) / cu
    u1 = jnp.maximum(
        aggu + sbl_iu_r[...] +
        jnp.dot(xu_ref[...], sWr_iu_r[...], preferred_element_type=f32), 0.0)
    ci_ = jnp.maximum(cnti_ref[0] + cnti_ref[1], 1.0)
    aggi = (acci_ref[0] + acci_ref[1]) / ci_
    i1 = jnp.maximum(
        aggi + sbl_ui_r[...] +
        jnp.dot(xi_ref[...], sWr_ui_r[...], preferred_element_type=f32), 0.0)
    hl_iu_r[...] = jnp.dot(i1, gWl_iu_r[...], preferred_element_type=f32) + gbl_iu_r[...]
    hr_iu_r[...] = jnp.dot(u1, gWr_iu_r[...], preferred_element_type=f32) + gbr_iu_r[...]
    hl_ui_r[...] = jnp.dot(u1, gWl_ui_r[...], preferred_element_type=f32) + gbl_ui_r[...]
    hr_ui_r[...] = jnp.dot(i1, gWr_ui_r[...], preferred_element_type=f32) + gbr_ui_r[...]

  row = pl.BlockSpec((BM, D), lambda b: (b, 0))
  two = pl.BlockSpec((2, BM, D), lambda b: (0, b, 0))
  cnt = pl.BlockSpec((2, BM, 1), lambda b: (0, b, 0))
  wsp = pl.BlockSpec((D, D), lambda b: (0, 0))
  bsp = pl.BlockSpec((1, D), lambda b: (0, 0))
  outs = [jax.ShapeDtypeStruct((NP, D), f32)] * 4
  return pl.pallas_call(
      body,
      grid=(NP // BM,),
      in_specs=[two, cnt, row, two, cnt, row,
                wsp, bsp, wsp, bsp,
                wsp, bsp, wsp, bsp,
                wsp, bsp, wsp, bsp],
      out_specs=[row, row, row, row],
      out_shape=outs,
  )(accu, cntu, xu, acci, cnti, xi,
    sWr_iu, sbl_iu, sWr_ui, sbl_ui,
    gWl_iu, gbl_iu, gWr_iu, gbr_iu,
    gWl_ui, gbl_ui, gWr_ui, gbr_ui)


def _exp_norm(l2):
  def body(l_ref, o_ref):
    l0 = l_ref[0]
    l1 = l_ref[1]
    o_ref[0] = jnp.exp(l0 - jnp.max(l0))
    o_ref[1] = jnp.exp(l1 - jnp.max(l1))
  r = EP // D
  return pl.pallas_call(
      body,
      out_shape=jax.ShapeDtypeStruct((2, r, D), f32),
  )(l2)


def _final(accu, denu, bu, acci, deni, bi):
  def body(au, du, bu_r, ai, di, bi_r, u2, i2):
    u2[...] = jnp.maximum(
        (au[0] + au[1]) / jnp.maximum(du[0] + du[1], 1e-16) + bu_r[...], 0.0)
    i2[...] = jnp.maximum(
        (ai[0] + ai[1]) / jnp.maximum(di[0] + di[1], 1e-16) + bi_r[...], 0.0)
  row = pl.BlockSpec((BM, D), lambda b: (b, 0))
  two = pl.BlockSpec((2, BM, D), lambda b: (0, b, 0))
  cnt = pl.BlockSpec((2, BM, 1), lambda b: (0, b, 0))
  bsp = pl.BlockSpec((1, D), lambda b: (0, 0))
  return pl.pallas_call(
      body,
      grid=(NP // BM,),
      in_specs=[two, cnt, bsp, two, cnt, bsp],
      out_specs=[row, row],
      out_shape=[jax.ShapeDtypeStruct((NP, D), f32)] * 2,
  )(accu, denu, bu, acci, deni, bi)


def _pad_rows(x):
  return jnp.concatenate([x, jnp.zeros((NP - N, D), f32)], axis=0)


def _pad_idx(v):
  return jnp.concatenate([v.astype(i32), jnp.full((EP - E,), PADN, i32)])


@jax.jit
def kernel(x_user, x_item, edge_index_ui, edge_index_iu,
           sage_ui_Wl, sage_ui_bl, sage_ui_Wr,
           sage_iu_Wl, sage_iu_bl, sage_iu_Wr,
           gat_ui_Wl, gat_ui_bl, gat_ui_Wr, gat_ui_br, gat_ui_att, gat_ui_b,
           gat_iu_Wl, gat_iu_bl, gat_iu_Wr, gat_iu_br, gat_iu_att, gat_iu_b):
  xu = _pad_rows(x_user)
  xi = _pad_rows(x_item)
  src_ui = _pad_idx(edge_index_ui[0])
  dst_ui = _pad_idx(edge_index_ui[1])
  src_iu = _pad_idx(edge_index_iu[0])
  dst_iu = _pad_idx(edge_index_iu[1])
  zeros_nd = jnp.zeros((NP, D), f32)
  dummy_w = jnp.zeros((EP,), f32)

  y2 = _mm2(jnp.stack([xu, xi]), jnp.stack([sage_ui_Wl, sage_iu_Wl]))
  acci, cnti = _seg_sum(y2[0], src_ui, dst_ui, dummy_w, zeros_nd)
  accu, cntu = _seg_sum(y2[1], src_iu, dst_iu, dummy_w, zeros_nd)

  hl_iu, hr_iu, hl_ui, hr_ui = _layer_mid(
      accu, cntu.reshape(NC, NP, 1), xu, acci, cnti.reshape(NC, NP, 1), xi,
      sage_iu_Wr, sage_iu_bl.reshape(1, D), sage_ui_Wr, sage_ui_bl.reshape(1, D),
      gat_iu_Wl, gat_iu_bl.reshape(1, D), gat_iu_Wr, gat_iu_br.reshape(1, D),
      gat_ui_Wl, gat_ui_bl.reshape(1, D), gat_ui_Wr, gat_ui_br.reshape(1, D))

  lo_iu = _logits(hl_iu, hr_iu, src_iu, dst_iu, gat_iu_att)
  lo_ui = _logits(hl_ui, hr_ui, src_ui, dst_ui, gat_ui_att)
  ex2 = _exp_norm(jnp.stack([lo_iu, lo_ui]).reshape(2, EP // D, D))
  ex_iu = ex2[0].reshape(EP)
  ex_ui = ex2[1].reshape(EP)

  waccu, wdenu = _seg_sum_w(hl_iu, src_iu, dst_iu, ex_iu, zeros_nd)
  wacci, wdeni = _seg_sum_w(hl_ui, src_ui, dst_ui, ex_ui, zeros_nd)

  u2, i2 = _final(waccu, wdenu.reshape(NC, NP, 1), gat_iu_b.reshape(1, D),
                  wacci, wdeni.reshape(NC, NP, 1), gat_ui_b.reshape(1, D))
  return (u2[:N], i2[:N])


# speedup vs baseline: 3.7886x; 1.2702x over previous
import functools

import jax
import jax.numpy as jnp
from jax import lax
from jax.experimental import pallas as pl
from jax.experimental.pallas import tpu as pltpu
from jax.experimental.pallas import tpu_sc as plsc

N = 10000
NP = 10240
D = 128
E = 320000
EP = 327680
NC = 2
NS = 16
NW = NC * NS
EW = EP // NW
CH = 128
NCH = EW // CH
NSL = NP // NS
PADN = NP - 1

f32 = jnp.float32
i32 = jnp.int32

_mesh = plsc.VectorSubcoreMesh(
    core_axis_name="c", subcore_axis_name="s", num_cores=NC, num_subcores=NS)


def _zero_vmem(ref, n):
  def b(k, _):
    ref[pl.ds(k * 16, 16)] = jnp.zeros((16,), f32)
    return _
  lax.fori_loop(0, n // 16, b, None)


NB = 2


def _make_seg_sum(weighted):
  def body(table, src, dst, w, zeros_nd,
           acc_out, hist_out,
           sidx0, sidx1, didx0, didx1, wv0, wv1, rows0, rows1, hist,
           acc_sh, ssem, gsem0, gsem1):
    sidxb = (sidx0, sidx1)
    didxb = (didx0, didx1)
    wvb = (wv0, wv1)
    rowsb = (rows0, rows1)
    gsems = (gsem0, gsem1)
    c = lax.axis_index("c")
    s = lax.axis_index("s")
    wid = c * NS + s
    pltpu.sync_copy(zeros_nd.at[pl.ds(s * NSL, NSL)],
                    acc_sh.at[pl.ds(s * NSL, NSL)])
    _zero_vmem(hist, NP)
    plsc.subcore_barrier()

    def group(g, __):
      gd = []
      for b in range(NB):
        base = wid * EW + (g * NB + b) * CH
        pltpu.sync_copy(src.at[pl.ds(base, CH)], sidxb[b])
        pltpu.sync_copy(dst.at[pl.ds(base, CH)], didxb[b])
        if weighted:
          pltpu.sync_copy(w.at[pl.ds(base, CH)], wvb[b])
        gd.append(pltpu.async_copy(table.at[sidxb[b]], rowsb[b], gsems[b]))
      sd = []
      for b in range(NB):
        gd[b].wait()
        if weighted:
          wrow = wvb[b]
          rows = rowsb[b]

          def _scale(e, ___):
            ws = plsc.load_gather(wrow, [jnp.full((16,), e, i32)])
            for k in range(D // 16):
              rows[e, pl.ds(k * 16, 16)] = rows[e, pl.ds(k * 16, 16)] * ws
            return ___
          lax.fori_loop(0, CH, _scale, None)
        sd.append(pltpu.async_copy(rowsb[b], acc_sh.at[didxb[b]],
                                   ssem, add=True))
        for j in range(CH // 16):
          dv = didxb[b][pl.ds(j * 16, 16)]
          if weighted:
            vals = wvb[b][pl.ds(j * 16, 16)]
          else:
            vals = jnp.ones((16,), f32)
          plsc.addupdate_scatter(hist, [dv], vals)
      for d in sd:
        d.wait()
      return __
    lax.fori_loop(0, NCH // NB, group, None)

    plsc.subcore_barrier()
    pltpu.sync_copy(acc_sh.at[pl.ds(s * NSL, NSL)],
                    acc_out.at[c, pl.ds(s * NSL, NSL)])
    pltpu.sync_copy(hist, hist_out.at[c, s])

  return pl.kernel(
      body,
      out_type=(jax.ShapeDtypeStruct((NC, NP, D), f32),
                jax.ShapeDtypeStruct((NC, NS, NP), f32)),
      mesh=_mesh,
      scratch_types=[
          pltpu.VMEM((CH,), i32),
          pltpu.VMEM((CH,), i32),
          pltpu.VMEM((CH,), i32),
          pltpu.VMEM((CH,), i32),
          pltpu.VMEM((CH,), f32),
          pltpu.VMEM((CH,), f32),
          pltpu.VMEM((CH, D), f32),
          pltpu.VMEM((CH, D), f32),
          pltpu.VMEM((NP,), f32),
          pltpu.VMEM_SHARED((NP, D), f32),
          pltpu.SemaphoreType.DMA,
          pltpu.SemaphoreType.DMA,
          pltpu.SemaphoreType.DMA,
      ],
      compiler_params=pltpu.CompilerParams(needs_layout_passes=False),
      name="sc_seg_sum_w" if weighted else "sc_seg_sum",
  )


_seg_sum = _make_seg_sum(False)
_seg_sum_w = _make_seg_sum(True)


NB2 = 2


def _logits_body(hl, hr, src, dst, att,
                 lo,
                 sidx0, sidx1, didx0, didx1, abuf0, abuf1, bbuf0, bbuf1,
                 attv, lbuf0, lbuf1, wsem, *gsems):
  sidxb = (sidx0, sidx1)
  didxb = (didx0, didx1)
  abufb = (abuf0, abuf1)
  bbufb = (bbuf0, bbuf1)
  lbufb = (lbuf0, lbuf1)
  c = lax.axis_index("c")
  s = lax.axis_index("s")
  wid = c * NS + s
  pltpu.sync_copy(att, attv)
  attk = [attv[pl.ds(k * 16, 16)] for k in range(D // 16)]
  lane0 = lax.iota(i32, 16) == 0

  def pair(g, _):
    gda = []
    gdb = []
    for b in range(NB2):
      base = wid * EW + (g * NB2 + b) * CH
      pltpu.sync_copy(src.at[pl.ds(base, CH)], sidxb[b])
      pltpu.sync_copy(dst.at[pl.ds(base, CH)], didxb[b])
      gda.append(pltpu.async_copy(hl.at[sidxb[b]], abufb[b], gsems[2 * b]))
      gdb.append(pltpu.async_copy(hr.at[didxb[b]], bbufb[b],
                                  gsems[2 * b + 1]))
    wd = []
    for b in range(NB2):
      base = wid * EW + (g * NB2 + b) * CH
      gda[b].wait()
      gdb[b].wait()
      abuf = abufb[b]
      bbuf = bbufb[b]
      lrow = lbufb[b]

      def _edge(e, ___):
        acc = jnp.zeros((16,), f32)
        for k in range(D // 16):
          t = abuf[e, pl.ds(k * 16, 16)] + bbuf[e, pl.ds(k * 16, 16)]
          lr = jnp.maximum(t, 0.2 * t)
          acc = acc + lr * attk[k]
        lg = jnp.sum(acc)
        plsc.store_scatter(lrow, [jnp.full((16,), e, i32)],
                           jnp.full((16,), lg, f32), mask=lane0)
        return ___
      lax.fori_loop(0, CH, _edge, None)
      wd.append(pltpu.async_copy(lrow, lo.at[pl.ds(base, CH)], wsem))
    for d in wd:
      d.wait()
    return _
  lax.fori_loop(0, NCH // NB2, pair, None)


_logits = pl.kernel(
    _logits_body,
    out_type=jax.ShapeDtypeStruct((EP,), f32),
    mesh=_mesh,
    scratch_types=[
        pltpu.VMEM((CH,), i32),
        pltpu.VMEM((CH,), i32),
        pltpu.VMEM((CH,), i32),
        pltpu.VMEM((CH,), i32),
        pltpu.VMEM((CH, D), f32),
        pltpu.VMEM((CH, D), f32),
        pltpu.VMEM((CH, D), f32),
        pltpu.VMEM((CH, D), f32),
        pltpu.VMEM((D,), f32),
        pltpu.VMEM((CH,), f32),
        pltpu.VMEM((CH,), f32),
        pltpu.SemaphoreType.DMA,
    ] + [pltpu.SemaphoreType.DMA] * (2 * NB2),
    compiler_params=pltpu.CompilerParams(needs_layout_passes=False),
    name="sc_gat_logits",
)


BM = 1024


def _mm2(x2, w2):
  def body(x_ref, w_ref, o_ref):
    o_ref[0] = jnp.dot(x_ref[0], w_ref[0], preferred_element_type=f32)
  return pl.pallas_call(
      body,
      grid=(2, NP // BM),
      in_specs=[pl.BlockSpec((1, BM, D), lambda a, b: (a, b, 0)),
                pl.BlockSpec((1, D, D), lambda a, b: (a, 0, 0))],
      out_specs=pl.BlockSpec((1, BM, D), lambda a, b: (a, b, 0)),
      out_shape=jax.ShapeDtypeStruct((2, NP, D), f32),
  )(x2, w2)


def _comb(h2):
  def body(h_ref, o_ref):
    o_ref[0] = jnp.sum(h_ref[0], axis=0)
    o_ref[1] = jnp.sum(h_ref[1], axis=0)
  return pl.pallas_call(
      body,
      out_shape=jax.ShapeDtypeStruct((2, NP), f32),
  )(h2)


def _layer_mid(accu, cntu, xu, acci, cnti, xi,
               sWr_iu, sbl_iu, sWr_ui, sbl_ui,
               gWl_iu, gbl_iu, gWr_iu, gbr_iu,
               gWl_ui, gbl_ui, gWr_ui, gbr_ui):
  def body(accu_ref, cntu_ref, xu_ref, acci_ref, cnti_ref, xi_ref,
           sWr_iu_r, sbl_iu_r, sWr_ui_r, sbl_ui_r,
           gWl_iu_r, gbl_iu_r, gWr_iu_r, gbr_iu_r,
           gWl_ui_r, gbl_ui_r, gWr_ui_r, gbr_ui_r,
           hl_iu_r, hr_iu_r, hl_ui_r, hr_ui_r):
    cu = jnp.maximum(cntu_ref[...], 1.0)
    aggu = (accu_ref[0] + accu_ref[1]) / cu
    u1 = jnp.maximum(
        aggu + sbl_iu_r[...] +
        jnp.dot(xu_ref[...], sWr_iu_r[...], preferred_element_type=f32), 0.0)
    ci_ = jnp.maximum(cnti_ref[...], 1.0)
    aggi = (acci_ref[0] + acci_ref[1]) / ci_
    i1 = jnp.maximum(
        aggi + sbl_ui_r[...] +
        jnp.dot(xi_ref[...], sWr_ui_r[...], preferred_element_type=f32), 0.0)
    hl_iu_r[...] = jnp.dot(i1, gWl_iu_r[...], preferred_element_type=f32) + gbl_iu_r[...]
    hr_iu_r[...] = jnp.dot(u1, gWr_iu_r[...], preferred_element_type=f32) + gbr_iu_r[...]
    hl_ui_r[...] = jnp.dot(u1, gWl_ui_r[...], preferred_element_type=f32) + gbl_ui_r[...]
    hr_ui_r[...] = jnp.dot(i1, gWr_ui_r[...], preferred_element_type=f32) + gbr_ui_r[...]

  row = pl.BlockSpec((BM, D), lambda b: (b, 0))
  two = pl.BlockSpec((2, BM, D), lambda b: (0, b, 0))
  cnt = pl.BlockSpec((BM, 1), lambda b: (b, 0))
  wsp = pl.BlockSpec((D, D), lambda b: (0, 0))
  bsp = pl.BlockSpec((1, D), lambda b: (0, 0))
  outs = [jax.ShapeDtypeStruct((NP, D), f32)] * 4
  return pl.pallas_call(
      body,
      grid=(NP // BM,),
      in_specs=[two, cnt, row, two, cnt, row,
                wsp, bsp, wsp, bsp,
                wsp, bsp, wsp, bsp,
                wsp, bsp, wsp, bsp],
      out_specs=[row, row, row, row],
      out_shape=outs,
  )(accu, cntu, xu, acci, cnti, xi,
    sWr_iu, sbl_iu, sWr_ui, sbl_ui,
    gWl_iu, gbl_iu, gWr_iu, gbr_iu,
    gWl_ui, gbl_ui, gWr_ui, gbr_ui)


def _exp_norm(l2):
  def body(l_ref, o_ref):
    l0 = l_ref[0]
    l1 = l_ref[1]
    o_ref[0] = jnp.exp(l0 - jnp.max(l0))
    o_ref[1] = jnp.exp(l1 - jnp.max(l1))
  r = EP // D
  return pl.pallas_call(
      body,
      out_shape=jax.ShapeDtypeStruct((2, r, D), f32),
  )(l2)


def _final(accu, denu, bu, acci, deni, bi):
  def body(au, du, bu_r, ai, di, bi_r, u2, i2):
    u2[...] = jnp.maximum(
        (au[0] + au[1]) / jnp.maximum(du[...], 1e-16) + bu_r[...], 0.0)
    i2[...] = jnp.maximum(
        (ai[0] + ai[1]) / jnp.maximum(di[...], 1e-16) + bi_r[...], 0.0)
  row = pl.BlockSpec((BM, D), lambda b: (b, 0))
  two = pl.BlockSpec((2, BM, D), lambda b: (0, b, 0))
  cnt = pl.BlockSpec((BM, 1), lambda b: (b, 0))
  bsp = pl.BlockSpec((1, D), lambda b: (0, 0))
  return pl.pallas_call(
      body,
      grid=(NP // BM,),
      in_specs=[two, cnt, bsp, two, cnt, bsp],
      out_specs=[row, row],
      out_shape=[jax.ShapeDtypeStruct((NP, D), f32)] * 2,
  )(accu, denu, bu, acci, deni, bi)


def _pad_rows(x):
  return jnp.concatenate([x, jnp.zeros((NP - N, D), f32)], axis=0)


def _pad_idx(v):
  return jnp.concatenate([v.astype(i32), jnp.full((EP - E,), PADN, i32)])


@jax.jit
def kernel(x_user, x_item, edge_index_ui, edge_index_iu,
           sage_ui_Wl, sage_ui_bl, sage_ui_Wr,
           sage_iu_Wl, sage_iu_bl, sage_iu_Wr,
           gat_ui_Wl, gat_ui_bl, gat_ui_Wr, gat_ui_br, gat_ui_att, gat_ui_b,
           gat_iu_Wl, gat_iu_bl, gat_iu_Wr, gat_iu_br, gat_iu_att, gat_iu_b):
  xu = _pad_rows(x_user)
  xi = _pad_rows(x_item)
  src_ui = _pad_idx(edge_index_ui[0])
  dst_ui = _pad_idx(edge_index_ui[1])
  src_iu = _pad_idx(edge_index_iu[0])
  dst_iu = _pad_idx(edge_index_iu[1])
  zeros_nd = jnp.zeros((NP, D), f32)
  dummy_w = jnp.zeros((EP,), f32)

  y2 = _mm2(jnp.stack([xu, xi]), jnp.stack([sage_ui_Wl, sage_iu_Wl]))
  acci, cnti = _seg_sum(y2[0], src_ui, dst_ui, dummy_w, zeros_nd)
  accu, cntu = _seg_sum(y2[1], src_iu, dst_iu, dummy_w, zeros_nd)
  cnt2 = _comb(jnp.stack([cntu.reshape(NW, NP), cnti.reshape(NW, NP)]))

  hl_iu, hr_iu, hl_ui, hr_ui = _layer_mid(
      accu, cnt2[0].reshape(NP, 1), xu, acci, cnt2[1].reshape(NP, 1), xi,
      sage_iu_Wr, sage_iu_bl.reshape(1, D), sage_ui_Wr, sage_ui_bl.reshape(1, D),
      gat_iu_Wl, gat_iu_bl.reshape(1, D), gat_iu_Wr, gat_iu_br.reshape(1, D),
      gat_ui_Wl, gat_ui_bl.reshape(1, D), gat_ui_Wr, gat_ui_br.reshape(1, D))

  lo_iu = _logits(hl_iu, hr_iu, src_iu, dst_iu, gat_iu_att)
  lo_ui = _logits(hl_ui, hr_ui, src_ui, dst_ui, gat_ui_att)
  ex2 = _exp_norm(jnp.stack([lo_iu, lo_ui]).reshape(2, EP // D, D))
  ex_iu = ex2[0].reshape(EP)
  ex_ui = ex2[1].reshape(EP)

  waccu, wdenu = _seg_sum_w(hl_iu, src_iu, dst_iu, ex_iu, zeros_nd)
  wacci, wdeni = _seg_sum_w(hl_ui, src_ui, dst_ui, ex_ui, zeros_nd)
  den2 = _comb(jnp.stack([wdenu.reshape(NW, NP), wdeni.reshape(NW, NP)]))

  u2, i2 = _final(waccu, den2[0].reshape(NP, 1), gat_iu_b.reshape(1, D),
                  wacci, den2[1].reshape(NP, 1), gat_ui_b.reshape(1, D))
  return (u2[:N], i2[:N])


# speedup vs baseline: 3.7906x; 1.0005x over previous
import functools

import jax
import jax.numpy as jnp
from jax import lax
from jax.experimental import pallas as pl
from jax.experimental.pallas import tpu as pltpu
from jax.experimental.pallas import tpu_sc as plsc

N = 10000
NP = 10240
D = 128
E = 320000
EP = 327680
NC = 2
NS = 16
NW = NC * NS
EW = EP // NW
CH = 128
NCH = EW // CH
NSL = NP // NS
PADN = NP - 1

f32 = jnp.float32
i32 = jnp.int32

_mesh = plsc.VectorSubcoreMesh(
    core_axis_name="c", subcore_axis_name="s", num_cores=NC, num_subcores=NS)


def _zero_vmem(ref, n):
  def b(k, _):
    ref[pl.ds(k * 16, 16)] = jnp.zeros((16,), f32)
    return _
  lax.fori_loop(0, n // 16, b, None)


NB = 2


def _make_seg_sum(weighted):
  def body(table, src, dst, w, zeros_nd,
           acc_out, hist_out,
           sidx0, sidx1, didx0, didx1, wv0, wv1, rows0, rows1, hist,
           acc_sh, ssem, gsem0, gsem1):
    sidxb = (sidx0, sidx1)
    didxb = (didx0, didx1)
    wvb = (wv0, wv1)
    rowsb = (rows0, rows1)
    gsems = (gsem0, gsem1)
    c = lax.axis_index("c")
    s = lax.axis_index("s")
    wid = c * NS + s
    pltpu.sync_copy(zeros_nd.at[pl.ds(s * NSL, NSL)],
                    acc_sh.at[pl.ds(s * NSL, NSL)])
    _zero_vmem(hist, NP)
    plsc.subcore_barrier()

    def group(g, __):
      gd = []
      for b in range(NB):
        base = wid * EW + (g * NB + b) * CH
        pltpu.sync_copy(src.at[pl.ds(base, CH)], sidxb[b])
        pltpu.sync_copy(dst.at[pl.ds(base, CH)], didxb[b])
        if weighted:
          pltpu.sync_copy(w.at[pl.ds(base, CH)], wvb[b])
        gd.append(pltpu.async_copy(table.at[sidxb[b]], rowsb[b], gsems[b]))
      sd = []
      for b in range(NB):
        gd[b].wait()
        if weighted:
          wrow = wvb[b]
          rows = rowsb[b]

          def _scale(g2, ___):
            for eo in range(2):
              e = g2 * 2 + eo
              ws = plsc.load_gather(wrow, [jnp.full((16,), e, i32)])
              for k in range(D // 16):
                rows[e, pl.ds(k * 16, 16)] = rows[e, pl.ds(k * 16, 16)] * ws
            return ___
          lax.fori_loop(0, CH // 2, _scale, None)
        sd.append(pltpu.async_copy(rowsb[b], acc_sh.at[didxb[b]],
                                   ssem, add=True))
        for j in range(CH // 16):
          dv = didxb[b][pl.ds(j * 16, 16)]
          if weighted:
            vals = wvb[b][pl.ds(j * 16, 16)]
          else:
            vals = jnp.ones((16,), f32)
          plsc.addupdate_scatter(hist, [dv], vals)
      for d in sd:
        d.wait()
      return __
    lax.fori_loop(0, NCH // NB, group, None)

    plsc.subcore_barrier()
    pltpu.sync_copy(acc_sh.at[pl.ds(s * NSL, NSL)],
                    acc_out.at[c, pl.ds(s * NSL, NSL)])
    pltpu.sync_copy(hist, hist_out.at[c, s])

  return pl.kernel(
      body,
      out_type=(jax.ShapeDtypeStruct((NC, NP, D), f32),
                jax.ShapeDtypeStruct((NC, NS, NP), f32)),
      mesh=_mesh,
      scratch_types=[
          pltpu.VMEM((CH,), i32),
          pltpu.VMEM((CH,), i32),
          pltpu.VMEM((CH,), i32),
          pltpu.VMEM((CH,), i32),
          pltpu.VMEM((CH,), f32),
          pltpu.VMEM((CH,), f32),
          pltpu.VMEM((CH, D), f32),
          pltpu.VMEM((CH, D), f32),
          pltpu.VMEM((NP,), f32),
          pltpu.VMEM_SHARED((NP, D), f32),
          pltpu.SemaphoreType.DMA,
          pltpu.SemaphoreType.DMA,
          pltpu.SemaphoreType.DMA,
      ],
      compiler_params=pltpu.CompilerParams(needs_layout_passes=False),
      name="sc_seg_sum_w" if weighted else "sc_seg_sum",
  )


_seg_sum = _make_seg_sum(False)
_seg_sum_w = _make_seg_sum(True)


NB2 = 2


def _logits_body(hl, hr, src, dst, att,
                 lo,
                 sidx0, sidx1, didx0, didx1, abuf0, abuf1, bbuf0, bbuf1,
                 attv, lbuf0, lbuf1, wsem, *gsems):
  sidxb = (sidx0, sidx1)
  didxb = (didx0, didx1)
  abufb = (abuf0, abuf1)
  bbufb = (bbuf0, bbuf1)
  lbufb = (lbuf0, lbuf1)
  c = lax.axis_index("c")
  s = lax.axis_index("s")
  wid = c * NS + s
  pltpu.sync_copy(att, attv)
  attk = [attv[pl.ds(k * 16, 16)] for k in range(D // 16)]
  lane0 = lax.iota(i32, 16) == 0

  def pair(g, _):
    gda = []
    gdb = []
    for b in range(NB2):
      base = wid * EW + (g * NB2 + b) * CH
      pltpu.sync_copy(src.at[pl.ds(base, CH)], sidxb[b])
      pltpu.sync_copy(dst.at[pl.ds(base, CH)], didxb[b])
      gda.append(pltpu.async_copy(hl.at[sidxb[b]], abufb[b], gsems[2 * b]))
      gdb.append(pltpu.async_copy(hr.at[didxb[b]], bbufb[b],
                                  gsems[2 * b + 1]))
    wd = []
    for b in range(NB2):
      base = wid * EW + (g * NB2 + b) * CH
      gda[b].wait()
      gdb[b].wait()
      abuf = abufb[b]
      bbuf = bbufb[b]
      lrow = lbufb[b]

      def _edge(g2, ___):
        for eo in range(2):
          e = g2 * 2 + eo
          acc = jnp.zeros((16,), f32)
          for k in range(D // 16):
            t = abuf[e, pl.ds(k * 16, 16)] + bbuf[e, pl.ds(k * 16, 16)]
            lr = jnp.maximum(t, 0.2 * t)
            acc = acc + lr * attk[k]
          lg = jnp.sum(acc)
          plsc.store_scatter(lrow, [jnp.full((16,), e, i32)],
                             jnp.full((16,), lg, f32), mask=lane0)
        return ___
      lax.fori_loop(0, CH // 2, _edge, None)
      wd.append(pltpu.async_copy(lrow, lo.at[pl.ds(base, CH)], wsem))
    for d in wd:
      d.wait()
    return _
  lax.fori_loop(0, NCH // NB2, pair, None)


_logits = pl.kernel(
    _logits_body,
    out_type=jax.ShapeDtypeStruct((EP,), f32),
    mesh=_mesh,
    scratch_types=[
        pltpu.VMEM((CH,), i32),
        pltpu.VMEM((CH,), i32),
        pltpu.VMEM((CH,), i32),
        pltpu.VMEM((CH,), i32),
        pltpu.VMEM((CH, D), f32),
        pltpu.VMEM((CH, D), f32),
        pltpu.VMEM((CH, D), f32),
        pltpu.VMEM((CH, D), f32),
        pltpu.VMEM((D,), f32),
        pltpu.VMEM((CH,), f32),
        pltpu.VMEM((CH,), f32),
        pltpu.SemaphoreType.DMA,
    ] + [pltpu.SemaphoreType.DMA] * (2 * NB2),
    compiler_params=pltpu.CompilerParams(needs_layout_passes=False),
    name="sc_gat_logits",
)


BM = 1024


def _mm2(x2, w2):
  def body(x_ref, w_ref, o_ref):
    o_ref[0] = jnp.dot(x_ref[0], w_ref[0], preferred_element_type=f32)
  return pl.pallas_call(
      body,
      grid=(2, NP // BM),
      in_specs=[pl.BlockSpec((1, BM, D), lambda a, b: (a, b, 0)),
                pl.BlockSpec((1, D, D), lambda a, b: (a, 0, 0))],
      out_specs=pl.BlockSpec((1, BM, D), lambda a, b: (a, b, 0)),
      out_shape=jax.ShapeDtypeStruct((2, NP, D), f32),
  )(x2, w2)


def _comb(h2):
  def body(h_ref, o_ref):
    o_ref[0] = jnp.sum(h_ref[0], axis=0)
    o_ref[1] = jnp.sum(h_ref[1], axis=0)
  return pl.pallas_call(
      body,
      out_shape=jax.ShapeDtypeStruct((2, NP), f32),
  )(h2)


def _layer_mid(accu, cntu, xu, acci, cnti, xi,
               sWr_iu, sbl_iu, sWr_ui, sbl_ui,
               gWl_iu, gbl_iu, gWr_iu, gbr_iu,
               gWl_ui, gbl_ui, gWr_ui, gbr_ui):
  def body(accu_ref, cntu_ref, xu_ref, acci_ref, cnti_ref, xi_ref,
           sWr_iu_r, sbl_iu_r, sWr_ui_r, sbl_ui_r,
           gWl_iu_r, gbl_iu_r, gWr_iu_r, gbr_iu_r,
           gWl_ui_r, gbl_ui_r, gWr_ui_r, gbr_ui_r,
           hl_iu_r, hr_iu_r, hl_ui_r, hr_ui_r):
    cu = jnp.maximum(cntu_ref[...], 1.0)
    aggu = (accu_ref[0] + accu_ref[1]) / cu
    u1 = jnp.maximum(
        aggu + sbl_iu_r[...] +
        jnp.dot(xu_ref[...], sWr_iu_r[...], preferred_element_type=f32), 0.0)
    ci_ = jnp.maximum(cnti_ref[...], 1.0)
    aggi = (acci_ref[0] + acci_ref[1]) / ci_
    i1 = jnp.maximum(
        aggi + sbl_ui_r[...] +
        jnp.dot(xi_ref[...], sWr_ui_r[...], preferred_element_type=f32), 0.0)
    hl_iu_r[...] = jnp.dot(i1, gWl_iu_r[...], preferred_element_type=f32) + gbl_iu_r[...]
    hr_iu_r[...] = jnp.dot(u1, gWr_iu_r[...], preferred_element_type=f32) + gbr_iu_r[...]
    hl_ui_r[...] = jnp.dot(u1, gWl_ui_r[...], preferred_element_type=f32) + gbl_ui_r[...]
    hr_ui_r[...] = jnp.dot(i1, gWr_ui_r[...], preferred_element_type=f32) + gbr_ui_r[...]

  row = pl.BlockSpec((BM, D), lambda b: (b, 0))
  two = pl.BlockSpec((2, BM, D), lambda b: (0, b, 0))
  cnt = pl.BlockSpec((BM, 1), lambda b: (b, 0))
  wsp = pl.BlockSpec((D, D), lambda b: (0, 0))
  bsp = pl.BlockSpec((1, D), lambda b: (0, 0))
  outs = [jax.ShapeDtypeStruct((NP, D), f32)] * 4
  return pl.pallas_call(
      body,
      grid=(NP // BM,),
      in_specs=[two, cnt, row, two, cnt, row,
                wsp, bsp, wsp, bsp,
                wsp, bsp, wsp, bsp,
                wsp, bsp, wsp, bsp],
      out_specs=[row, row, row, row],
      out_shape=outs,
  )(accu, cntu, xu, acci, cnti, xi,
    sWr_iu, sbl_iu, sWr_ui, sbl_ui,
    gWl_iu, gbl_iu, gWr_iu, gbr_iu,
    gWl_ui, gbl_ui, gWr_ui, gbr_ui)


def _exp_norm(l2):
  def body(l_ref, o_ref):
    l0 = l_ref[0]
    l1 = l_ref[1]
    o_ref[0] = jnp.exp(l0 - jnp.max(l0))
    o_ref[1] = jnp.exp(l1 - jnp.max(l1))
  r = EP // D
  return pl.pallas_call(
      body,
      out_shape=jax.ShapeDtypeStruct((2, r, D), f32),
  )(l2)


def _final(accu, denu, bu, acci, deni, bi):
  def body(au, du, bu_r, ai, di, bi_r, u2, i2):
    u2[...] = jnp.maximum(
        (au[0] + au[1]) / jnp.maximum(du[...], 1e-16) + bu_r[...], 0.0)
    i2[...] = jnp.maximum(
        (ai[0] + ai[1]) / jnp.maximum(di[...], 1e-16) + bi_r[...], 0.0)
  row = pl.BlockSpec((BM, D), lambda b: (b, 0))
  two = pl.BlockSpec((2, BM, D), lambda b: (0, b, 0))
  cnt = pl.BlockSpec((BM, 1), lambda b: (b, 0))
  bsp = pl.BlockSpec((1, D), lambda b: (0, 0))
  return pl.pallas_call(
      body,
      grid=(NP // BM,),
      in_specs=[two, cnt, bsp, two, cnt, bsp],
      out_specs=[row, row],
      out_shape=[jax.ShapeDtypeStruct((NP, D), f32)] * 2,
  )(accu, denu, bu, acci, deni, bi)


def _pad_rows(x):
  return jnp.concatenate([x, jnp.zeros((NP - N, D), f32)], axis=0)


def _pad_idx(v):
  return jnp.concatenate([v.astype(i32), jnp.full((EP - E,), PADN, i32)])


@jax.jit
def kernel(x_user, x_item, edge_index_ui, edge_index_iu,
           sage_ui_Wl, sage_ui_bl, sage_ui_Wr,
           sage_iu_Wl, sage_iu_bl, sage_iu_Wr,
           gat_ui_Wl, gat_ui_bl, gat_ui_Wr, gat_ui_br, gat_ui_att, gat_ui_b,
           gat_iu_Wl, gat_iu_bl, gat_iu_Wr, gat_iu_br, gat_iu_att, gat_iu_b):
  xu = _pad_rows(x_user)
  xi = _pad_rows(x_item)
  src_ui = _pad_idx(edge_index_ui[0])
  dst_ui = _pad_idx(edge_index_ui[1])
  src_iu = _pad_idx(edge_index_iu[0])
  dst_iu = _pad_idx(edge_index_iu[1])
  zeros_nd = jnp.zeros((NP, D), f32)
  dummy_w = jnp.zeros((EP,), f32)

  y2 = _mm2(jnp.stack([xu, xi]), jnp.stack([sage_ui_Wl, sage_iu_Wl]))
  acci, cnti = _seg_sum(y2[0], src_ui, dst_ui, dummy_w, zeros_nd)
  accu, cntu = _seg_sum(y2[1], src_iu, dst_iu, dummy_w, zeros_nd)
  cnt2 = _comb(jnp.stack([cntu.reshape(NW, NP), cnti.reshape(NW, NP)]))

  hl_iu, hr_iu, hl_ui, hr_ui = _layer_mid(
      accu, cnt2[0].reshape(NP, 1), xu, acci, cnt2[1].reshape(NP, 1), xi,
      sage_iu_Wr, sage_iu_bl.reshape(1, D), sage_ui_Wr, sage_ui_bl.reshape(1, D),
      gat_iu_Wl, gat_iu_bl.reshape(1, D), gat_iu_Wr, gat_iu_br.reshape(1, D),
      gat_ui_Wl, gat_ui_bl.reshape(1, D), gat_ui_Wr, gat_ui_br.reshape(1, D))

  lo_iu = _logits(hl_iu, hr_iu, src_iu, dst_iu, gat_iu_att)
  lo_ui = _logits(hl_ui, hr_ui, src_ui, dst_ui, gat_ui_att)
  ex2 = _exp_norm(jnp.stack([lo_iu, lo_ui]).reshape(2, EP // D, D))
  ex_iu = ex2[0].reshape(EP)
  ex_ui = ex2[1].reshape(EP)

  waccu, wdenu = _seg_sum_w(hl_iu, src_iu, dst_iu, ex_iu, zeros_nd)
  wacci, wdeni = _seg_sum_w(hl_ui, src_ui, dst_ui, ex_ui, zeros_nd)
  den2 = _comb(jnp.stack([wdenu.reshape(NW, NP), wdeni.reshape(NW, NP)]))

  u2, i2 = _final(waccu, den2[0].reshape(NP, 1), gat_iu_b.reshape(1, D),
                  wacci, den2[1].reshape(NP, 1), gat_ui_b.reshape(1, D))
  return (u2[:N], i2[:N])


# speedup vs baseline: 5.4897x; 1.4483x over previous
import functools

import jax
import jax.numpy as jnp
from jax import lax
from jax.experimental import pallas as pl
from jax.experimental.pallas import tpu as pltpu
from jax.experimental.pallas import tpu_sc as plsc

N = 10000
NP = 10240
D = 128
E = 320000
EP = 327680
NC = 2
NS = 16
NW = NC * NS
EW = EP // NW
CH = 128
NCH = EW // CH
NSL = NP // NS
PADN = NP - 1

f32 = jnp.float32
i32 = jnp.int32

_mesh = plsc.VectorSubcoreMesh(
    core_axis_name="c", subcore_axis_name="s", num_cores=NC, num_subcores=NS)


def _zero_vmem(ref, n):
  def b(k, _):
    ref[pl.ds(k * 16, 16)] = jnp.zeros((16,), f32)
    return _
  lax.fori_loop(0, n // 16, b, None)


NB = 2
EW2 = EP // NS
NCH2 = EW2 // CH


def _make_seg_sum(weighted):
  def body(table, src, dst, w, zeros_nd,
           acc_out, hist_out,
           sidx0, sidx1, didx0, didx1, wv0, wv1, rows0, rows1, hist,
           acc_sh, ssem, gsem0, gsem1):
    sidxb = (sidx0, sidx1)
    didxb = (didx0, didx1)
    wvb = (wv0, wv1)
    rowsb = (rows0, rows1)
    gsems = (gsem0, gsem1)
    c = lax.axis_index("c")
    s = lax.axis_index("s")
    pltpu.sync_copy(zeros_nd.at[pl.ds(s * NSL, NSL)],
                    acc_sh.at[pl.ds(s * NSL, NSL)])
    _zero_vmem(hist, NP)
    plsc.subcore_barrier()

    def group(g, __):
      gd = []
      for b in range(NB):
        base = c * EP + s * EW2 + (g * NB + b) * CH
        pltpu.sync_copy(src.at[pl.ds(base, CH)], sidxb[b])
        pltpu.sync_copy(dst.at[pl.ds(base, CH)], didxb[b])
        if weighted:
          pltpu.sync_copy(w.at[pl.ds(base, CH)], wvb[b])
        gd.append(pltpu.async_copy(table.at[sidxb[b]], rowsb[b], gsems[b]))
      sd = []
      for b in range(NB):
        gd[b].wait()
        if weighted:
          wrow = wvb[b]
          rows = rowsb[b]

          def _scale(g2, ___):
            for eo in range(2):
              e = g2 * 2 + eo
              ws = plsc.load_gather(wrow, [jnp.full((16,), e, i32)])
              for k in range(D // 16):
                rows[e, pl.ds(k * 16, 16)] = rows[e, pl.ds(k * 16, 16)] * ws
            return ___
          lax.fori_loop(0, CH // 2, _scale, None)
        sd.append(pltpu.async_copy(rowsb[b], acc_sh.at[didxb[b]],
                                   ssem, add=True))
        for j in range(CH // 16):
          dv = didxb[b][pl.ds(j * 16, 16)]
          if weighted:
            vals = wvb[b][pl.ds(j * 16, 16)]
          else:
            vals = jnp.ones((16,), f32)
          plsc.addupdate_scatter(hist, [dv], vals)
      for d in sd:
        d.wait()
      return __
    lax.fori_loop(0, NCH2 // NB, group, None)

    plsc.subcore_barrier()
    pltpu.sync_copy(acc_sh.at[pl.ds(s * NSL, NSL)],
                    acc_out.at[c, pl.ds(s * NSL, NSL)])
    pltpu.sync_copy(hist, hist_out.at[c, s])

  return pl.kernel(
      body,
      out_type=(jax.ShapeDtypeStruct((NC, NP, D), f32),
                jax.ShapeDtypeStruct((NC, NS, NP), f32)),
      mesh=_mesh,
      scratch_types=[
          pltpu.VMEM((CH,), i32),
          pltpu.VMEM((CH,), i32),
          pltpu.VMEM((CH,), i32),
          pltpu.VMEM((CH,), i32),
          pltpu.VMEM((CH,), f32),
          pltpu.VMEM((CH,), f32),
          pltpu.VMEM((CH, D), f32),
          pltpu.VMEM((CH, D), f32),
          pltpu.VMEM((NP,), f32),
          pltpu.VMEM_SHARED((NP, D), f32),
          pltpu.SemaphoreType.DMA,
          pltpu.SemaphoreType.DMA,
          pltpu.SemaphoreType.DMA,
      ],
      compiler_params=pltpu.CompilerParams(needs_layout_passes=False),
      name="sc_seg_sum_w" if weighted else "sc_seg_sum",
  )


_seg_sum = _make_seg_sum(False)
_seg_sum_w = _make_seg_sum(True)


NB2 = 2


def _logits_body(hl, hr, src, dst, att,
                 lo,
                 sidx0, sidx1, didx0, didx1, abuf0, abuf1, bbuf0, bbuf1,
                 attv, lbuf0, lbuf1, wsem, *gsems):
  sidxb = (sidx0, sidx1)
  didxb = (didx0, didx1)
  abufb = (abuf0, abuf1)
  bbufb = (bbuf0, bbuf1)
  lbufb = (lbuf0, lbuf1)
  c = lax.axis_index("c")
  s = lax.axis_index("s")
  pltpu.sync_copy(att.at[c], attv)
  attk = [attv[pl.ds(k * 16, 16)] for k in range(D // 16)]
  lane0 = lax.iota(i32, 16) == 0

  def pair(g, _):
    gda = []
    gdb = []
    for b in range(NB2):
      base = c * EP + s * EW2 + (g * NB2 + b) * CH
      pltpu.sync_copy(src.at[pl.ds(base, CH)], sidxb[b])
      pltpu.sync_copy(dst.at[pl.ds(base, CH)], didxb[b])
      gda.append(pltpu.async_copy(hl.at[sidxb[b]], abufb[b], gsems[2 * b]))
      gdb.append(pltpu.async_copy(hr.at[didxb[b]], bbufb[b],
                                  gsems[2 * b + 1]))
    wd = []
    for b in range(NB2):
      base = c * EP + s * EW2 + (g * NB2 + b) * CH
      gda[b].wait()
      gdb[b].wait()
      abuf = abufb[b]
      bbuf = bbufb[b]
      lrow = lbufb[b]

      def _edge(g2, ___):
        for eo in range(2):
          e = g2 * 2 + eo
          acc = jnp.zeros((16,), f32)
          for k in range(D // 16):
            t = abuf[e, pl.ds(k * 16, 16)] + bbuf[e, pl.ds(k * 16, 16)]
            lr = jnp.maximum(t, 0.2 * t)
            acc = acc + lr * attk[k]
          lg = jnp.sum(acc)
          plsc.store_scatter(lrow, [jnp.full((16,), e, i32)],
                             jnp.full((16,), lg, f32), mask=lane0)
        return ___
      lax.fori_loop(0, CH // 2, _edge, None)
      wd.append(pltpu.async_copy(lrow, lo.at[pl.ds(base, CH)], wsem))
    for d in wd:
      d.wait()
    return _
  lax.fori_loop(0, NCH2 // NB2, pair, None)


_logits = pl.kernel(
    _logits_body,
    out_type=jax.ShapeDtypeStruct((2 * EP,), f32),
    mesh=_mesh,
    scratch_types=[
        pltpu.VMEM((CH,), i32),
        pltpu.VMEM((CH,), i32),
        pltpu.VMEM((CH,), i32),
        pltpu.VMEM((CH,), i32),
        pltpu.VMEM((CH, D), f32),
        pltpu.VMEM((CH, D), f32),
        pltpu.VMEM((CH, D), f32),
        pltpu.VMEM((CH, D), f32),
        pltpu.VMEM((D,), f32),
        pltpu.VMEM((CH,), f32),
        pltpu.VMEM((CH,), f32),
        pltpu.SemaphoreType.DMA,
    ] + [pltpu.SemaphoreType.DMA] * (2 * NB2),
    compiler_params=pltpu.CompilerParams(needs_layout_passes=False),
    name="sc_gat_logits",
)


BM = 1024


def _mm2(x2, w2):
  def body(x_ref, w_ref, o_ref):
    o_ref[0] = jnp.dot(x_ref[0], w_ref[0], preferred_element_type=f32)
  return pl.pallas_call(
      body,
      grid=(2, NP // BM),
      in_specs=[pl.BlockSpec((1, BM, D), lambda a, b: (a, b, 0)),
                pl.BlockSpec((1, D, D), lambda a, b: (a, 0, 0))],
      out_specs=pl.BlockSpec((1, BM, D), lambda a, b: (a, b, 0)),
      out_shape=jax.ShapeDtypeStruct((2, NP, D), f32),
  )(x2, w2)


def _comb(h2):
  def body(h_ref, o_ref):
    o_ref[0] = jnp.sum(h_ref[0], axis=0)
    o_ref[1] = jnp.sum(h_ref[1], axis=0)
  return pl.pallas_call(
      body,
      out_shape=jax.ShapeDtypeStruct((2, NP), f32),
  )(h2)


def _layer_mid(accu, cntu, xu, acci, cnti, xi,
               sWr_iu, sbl_iu, sWr_ui, sbl_ui,
               gWl_iu, gbl_iu, gWr_iu, gbr_iu,
               gWl_ui, gbl_ui, gWr_ui, gbr_ui):
  def body(accu_ref, cntu_ref, xu_ref, acci_ref, cnti_ref, xi_ref,
           sWr_iu_r, sbl_iu_r, sWr_ui_r, sbl_ui_r,
           gWl_iu_r, gbl_iu_r, gWr_iu_r, gbr_iu_r,
           gWl_ui_r, gbl_ui_r, gWr_ui_r, gbr_ui_r,
           hl_iu_r, hr_iu_r, hl_ui_r, hr_ui_r):
    cu = jnp.maximum(cntu_ref[...], 1.0)
    aggu = accu_ref[...] / cu
    u1 = jnp.maximum(
        aggu + sbl_iu_r[...] +
        jnp.dot(xu_ref[...], sWr_iu_r[...], preferred_element_type=f32), 0.0)
    ci_ = jnp.maximum(cnti_ref[...], 1.0)
    aggi = acci_ref[...] / ci_
    i1 = jnp.maximum(
        aggi + sbl_ui_r[...] +
        jnp.dot(xi_ref[...], sWr_ui_r[...], preferred_element_type=f32), 0.0)
    hl_iu_r[...] = jnp.dot(i1, gWl_iu_r[...], preferred_element_type=f32) + gbl_iu_r[...]
    hr_iu_r[...] = jnp.dot(u1, gWr_iu_r[...], preferred_element_type=f32) + gbr_iu_r[...]
    hl_ui_r[...] = jnp.dot(u1, gWl_ui_r[...], preferred_element_type=f32) + gbl_ui_r[...]
    hr_ui_r[...] = jnp.dot(i1, gWr_ui_r[...], preferred_element_type=f32) + gbr_ui_r[...]

  row = pl.BlockSpec((BM, D), lambda b: (b, 0))
  cnt = pl.BlockSpec((BM, 1), lambda b: (b, 0))
  wsp = pl.BlockSpec((D, D), lambda b: (0, 0))
  bsp = pl.BlockSpec((1, D), lambda b: (0, 0))
  outs = [jax.ShapeDtypeStruct((NP, D), f32)] * 4
  return pl.pallas_call(
      body,
      grid=(NP // BM,),
      in_specs=[row, cnt, row, row, cnt, row,
                wsp, bsp, wsp, bsp,
                wsp, bsp, wsp, bsp,
                wsp, bsp, wsp, bsp],
      out_specs=[row, row, row, row],
      out_shape=outs,
  )(accu, cntu, xu, acci, cnti, xi,
    sWr_iu, sbl_iu, sWr_ui, sbl_ui,
    gWl_iu, gbl_iu, gWr_iu, gbr_iu,
    gWl_ui, gbl_ui, gWr_ui, gbr_ui)


def _exp_norm(l2):
  def body(l_ref, o_ref):
    l0 = l_ref[0]
    l1 = l_ref[1]
    o_ref[0] = jnp.exp(l0 - jnp.max(l0))
    o_ref[1] = jnp.exp(l1 - jnp.max(l1))
  r = EP // D
  return pl.pallas_call(
      body,
      out_shape=jax.ShapeDtypeStruct((2, r, D), f32),
  )(l2)


def _final(accu, denu, bu, acci, deni, bi):
  def body(au, du, bu_r, ai, di, bi_r, u2, i2):
    u2[...] = jnp.maximum(
        au[...] / jnp.maximum(du[...], 1e-16) + bu_r[...], 0.0)
    i2[...] = jnp.maximum(
        ai[...] / jnp.maximum(di[...], 1e-16) + bi_r[...], 0.0)
  row = pl.BlockSpec((BM, D), lambda b: (b, 0))
  cnt = pl.BlockSpec((BM, 1), lambda b: (b, 0))
  bsp = pl.BlockSpec((1, D), lambda b: (0, 0))
  return pl.pallas_call(
      body,
      grid=(NP // BM,),
      in_specs=[row, cnt, bsp, row, cnt, bsp],
      out_specs=[row, row],
      out_shape=[jax.ShapeDtypeStruct((NP, D), f32)] * 2,
  )(accu, denu, bu, acci, deni, bi)


def _pad_rows(x):
  return jnp.concatenate([x, jnp.zeros((NP - N, D), f32)], axis=0)


def _pad_idx(v):
  return jnp.concatenate([v.astype(i32), jnp.full((EP - E,), PADN, i32)])


@jax.jit
def kernel(x_user, x_item, edge_index_ui, edge_index_iu,
           sage_ui_Wl, sage_ui_bl, sage_ui_Wr,
           sage_iu_Wl, sage_iu_bl, sage_iu_Wr,
           gat_ui_Wl, gat_ui_bl, gat_ui_Wr, gat_ui_br, gat_ui_att, gat_ui_b,
           gat_iu_Wl, gat_iu_bl, gat_iu_Wr, gat_iu_br, gat_iu_att, gat_iu_b):
  xu = _pad_rows(x_user)
  xi = _pad_rows(x_item)
  src_ui = _pad_idx(edge_index_ui[0])
  dst_ui = _pad_idx(edge_index_ui[1])
  src_iu = _pad_idx(edge_index_iu[0])
  dst_iu = _pad_idx(edge_index_iu[1])
  zeros_nd = jnp.zeros((NP, D), f32)
  dummy_w = jnp.zeros((2 * EP,), f32)

  src2 = jnp.concatenate([src_ui, src_iu + NP])
  dstg2 = jnp.concatenate([dst_ui, dst_iu + NP])
  dsts2 = jnp.concatenate([dst_ui, dst_iu])

  y2 = _mm2(jnp.stack([xu, xi]), jnp.stack([sage_ui_Wl, sage_iu_Wl]))
  acc2, hist2 = _seg_sum(y2.reshape(2 * NP, D), src2, dsts2, dummy_w,
                         zeros_nd)
  cnt2 = _comb(hist2)
  acci, accu = acc2[0], acc2[1]

  hl_iu, hr_iu, hl_ui, hr_ui = _layer_mid(
      accu, cnt2[1].reshape(NP, 1), xu, acci, cnt2[0].reshape(NP, 1), xi,
      sage_iu_Wr, sage_iu_bl.reshape(1, D), sage_ui_Wr, sage_ui_bl.reshape(1, D),
      gat_iu_Wl, gat_iu_bl.reshape(1, D), gat_iu_Wr, gat_iu_br.reshape(1, D),
      gat_ui_Wl, gat_ui_bl.reshape(1, D), gat_ui_Wr, gat_ui_br.reshape(1, D))

  hl2 = jnp.concatenate([hl_ui, hl_iu])
  hr2 = jnp.concatenate([hr_ui, hr_iu])
  att2 = jnp.stack([gat_ui_att, gat_iu_att])
  lo2 = _logits(hl2, hr2, src2, dstg2, att2)
  ex2 = _exp_norm(lo2.reshape(2, EP // D, D)).reshape(2 * EP)

  wacc2, whist2 = _seg_sum_w(hl2, src2, dsts2, ex2, zeros_nd)
  den2 = _comb(whist2)

  u2, i2 = _final(wacc2[1], den2[1].reshape(NP, 1), gat_iu_b.reshape(1, D),
                  wacc2[0], den2[0].reshape(NP, 1), gat_ui_b.reshape(1, D))
  return (u2[:N], i2[:N])
